# baseline (speedup 1.0000x reference)
import functools

import jax
import jax.numpy as jnp
from jax import lax
from jax.experimental import pallas as pl
from jax.experimental.pallas import tpu as pltpu

_LANES = 128
_N_CORES = 2
_RING = 3


def _mean_from(load, L, n_full, tail, inv_l):
    if n_full == 0:
        s = jnp.sum(load(0, L).astype(jnp.float32), axis=-1, keepdims=True)
    else:
        acc = load(0, _LANES).astype(jnp.float32)
        for c in range(1, n_full):
            acc = acc + load(c * _LANES, (c + 1) * _LANES).astype(jnp.float32)
        s = jnp.sum(acc, axis=-1, keepdims=True)
        if tail:
            s = s + jnp.sum(load(n_full * _LANES, L).astype(jnp.float32),
                            axis=-1, keepdims=True)
    return s * jnp.float32(inv_l)


def _ring_body(x_hbm, o_ref, bufs, sems, *, tbc, n_steps, L, n_full, tail,
               inv_l, rows_per_core):
    base = pl.program_id(0) * rows_per_core

    def start(step):
        slot = step % _RING
        pltpu.make_async_copy(
            x_hbm.at[pl.ds(base + step * tbc, tbc), :],
            bufs.at[slot], sems.at[slot]).start()

    def wait(step):
        slot = step % _RING
        pltpu.make_async_copy(
            bufs.at[slot], bufs.at[slot], sems.at[slot]).wait()

    for s in range(min(_RING, n_steps)):
        start(s)
    for s in range(n_steps):
        wait(s)
        if s + _RING < n_steps:
            start(s + _RING)
        slot = s % _RING
        m = _mean_from(lambda lo, hi: bufs[slot, :, lo:hi],
                       L, n_full, tail, inv_l)
        o_ref[s * tbc:(s + 1) * tbc, :] = m.astype(o_ref.dtype)


def _block_body(x_ref, o_ref, *, L, n_full, tail, inv_l):
    m = _mean_from(lambda lo, hi: x_ref[:, lo:hi], L, n_full, tail, inv_l)
    o_ref[...] = m.astype(o_ref.dtype)


def _pick_chunk_rows(rows_per_core, row_bytes, target_bytes=8 << 20):
    tbc = max(8, min(rows_per_core, (target_bytes // row_bytes) // 8 * 8))
    while tbc > 8 and rows_per_core % tbc != 0:
        tbc -= 8
    return tbc if rows_per_core % tbc == 0 else 0


def kernel(x):
    shape = x.shape
    L = shape[-1]
    lead = shape[:-1]
    B = 1
    for d in lead:
        B *= d
    x2 = x.reshape(B, L)

    itemsize = jnp.dtype(x.dtype).itemsize
    row_bytes = L * itemsize
    n_full = L // _LANES
    tail = L % _LANES != 0
    inv_l = 1.0 / L

    rows_per_core = B // _N_CORES
    tbc = (_pick_chunk_rows(rows_per_core, row_bytes)
           if B % _N_CORES == 0 and rows_per_core >= 16 else 0)

    if tbc and rows_per_core // tbc >= 2 and tbc * row_bytes <= (8 << 20):
        n_steps = rows_per_core // tbc
        vlim = int(min(max(_RING * tbc * row_bytes + (8 << 20), 16 << 20),
                       48 << 20))
        out = pl.pallas_call(
            functools.partial(_ring_body, tbc=tbc, n_steps=n_steps, L=L,
                              n_full=n_full, tail=tail, inv_l=inv_l,
                              rows_per_core=rows_per_core),
            out_shape=jax.ShapeDtypeStruct((B, 1), x.dtype),
            grid=(_N_CORES,),
            in_specs=[pl.BlockSpec(memory_space=pl.ANY)],
            out_specs=pl.BlockSpec((rows_per_core, 1), lambda c: (c, 0)),
            scratch_shapes=[pltpu.VMEM((_RING, tbc, L), x.dtype),
                            pltpu.SemaphoreType.DMA((_RING,))],
            compiler_params=pltpu.CompilerParams(
                dimension_semantics=("parallel",),
                vmem_limit_bytes=vlim),
        )(x2)
    else:
        TB = max(8, min(1024, ((8 << 20) // row_bytes) // 8 * 8))
        while TB > 8 and -(-B // TB) < 2:
            TB //= 2
        grid_b = -(-B // TB)
        vlim = int(min(max(4 * TB * row_bytes + (4 << 20), 16 << 20),
                       48 << 20))
        out = pl.pallas_call(
            functools.partial(_block_body, L=L, n_full=n_full, tail=tail,
                              inv_l=inv_l),
            out_shape=jax.ShapeDtypeStruct((B, 1), x.dtype),
            grid=(grid_b,),
            in_specs=[pl.BlockSpec((TB, L), lambda b: (b, 0))],
            out_specs=pl.BlockSpec((TB, 1), lambda b: (b, 0)),
            compiler_params=pltpu.CompilerParams(
                dimension_semantics=("parallel",),
                vmem_limit_bytes=vlim),
        )(x2)

    return out.reshape(lead)


# speedup vs baseline: 1.0611x; 1.0611x over previous
import functools

import jax
import jax.numpy as jnp
from jax.experimental import pallas as pl
from jax.experimental.pallas import tpu as pltpu

_LANES = 128


def _pool_body(x_ref, o_ref, *, L, n_full, tail, inv_l):
    if n_full == 0:
        s = jnp.sum(x_ref[...].astype(jnp.float32), axis=-1, keepdims=True)
    else:
        acc = x_ref[:, 0:_LANES].astype(jnp.float32)
        for c in range(1, n_full):
            acc = acc + x_ref[:, c * _LANES:(c + 1) * _LANES].astype(jnp.float32)
        s = jnp.sum(acc, axis=-1, keepdims=True)
        if tail:
            t = x_ref[:, n_full * _LANES:L].astype(jnp.float32)
            s = s + jnp.sum(t, axis=-1, keepdims=True)
    o_ref[...] = (s * jnp.float32(inv_l)).astype(o_ref.dtype)


def _pick_tb(B, row_bytes, target_bytes=8 << 20):
    tb = max(8, min(1024, (target_bytes // row_bytes) // 8 * 8))
    while tb > 8 and -(-B // tb) < 2:
        tb //= 2
    return tb


def kernel(x):
    shape = x.shape
    L = shape[-1]
    lead = shape[:-1]
    B = 1
    for d in lead:
        B *= d
    x2 = x.reshape(B, L)

    itemsize = jnp.dtype(x.dtype).itemsize
    row_bytes = L * itemsize
    TB = _pick_tb(B, row_bytes)
    grid_b = -(-B // TB)

    n_full = L // _LANES
    tail = L % _LANES != 0

    vlim = int(min(max(4 * TB * row_bytes + (4 << 20), 16 << 20), 48 << 20))
    out = pl.pallas_call(
        functools.partial(_pool_body, L=L, n_full=n_full, tail=tail,
                          inv_l=1.0 / L),
        out_shape=jax.ShapeDtypeStruct((B, 1), x.dtype),
        grid=(grid_b,),
        in_specs=[pl.BlockSpec((TB, L), lambda b: (b, 0))],
        out_specs=pl.BlockSpec((TB, 1), lambda b: (b, 0)),
        compiler_params=pltpu.CompilerParams(
            dimension_semantics=("parallel",),
            vmem_limit_bytes=vlim),
    )(x2)

    return out.reshape(lead)
